# Initial kernel scaffold; baseline (speedup 1.0000x reference)
import jax
import jax.numpy as jnp
from jax.experimental import pallas as pl


def kernel(features, edge_index, W1, b1, W2, b2):
    raise NotImplementedError("write your pallas kernel here")


# speedup vs baseline: 5.4339x; 5.4339x over previous
import jax
import jax.numpy as jnp
from jax import lax
from jax.experimental import pallas as pl
from jax.experimental.pallas import tpu as pltpu
from jax.experimental.pallas import tpu_sc as plsc

N = 50000
E = 800000
D = 64
K = 10
ALPHA = 0.1

NC = 2
NS = 16
NW = NC * NS
B = 128
NBATCH = 6400
E_PAD = NBATCH * B
BPT = NBATCH // NS
BPT_DEG = NBATCH // NW
N_PAD = 50176
RPT = N_PAD // NS
UB = 112
NUB = RPT // UB
ZCH = 392

_f32 = jnp.float32
_i32 = jnp.int32


def _prep_body(x_ref, w1_ref, b1_ref, w2_ref, b2_ref, degw_ref,
               u0_ref, init0_ref, c1e_ref, c2e_ref):
    x = x_ref[...]
    h = jnp.dot(x, w1_ref[...].T, preferred_element_type=_f32) + b1_ref[...]
    h = jnp.maximum(h, 0.0)
    h0 = jnp.dot(h, w2_ref[...].T, preferred_element_type=_f32) + b2_ref[...]
    degw = degw_ref[...]
    deg = degw[0, :, 0] + degw[1, :, 0]
    deg = jnp.maximum(deg, 1.0)
    norm = lax.rsqrt(deg)[:, None]
    u0_ref[...] = h0 * norm
    init0_ref[...] = (ALPHA / (1.0 - ALPHA)) * h0 / norm
    nb = x.shape[0]
    c1e_ref[...] = jnp.broadcast_to((1.0 - ALPHA) * norm * norm, (nb, 32))
    c2e_ref[...] = jnp.broadcast_to((1.0 - ALPHA) * norm, (nb, 32))


def _tc_prep(features, W1, b1, W2, b2, degw):
    nb = 448
    grid = N_PAD // nb
    return pl.pallas_call(
        _prep_body,
        grid=(grid,),
        in_specs=[
            pl.BlockSpec((nb, D), lambda i: (i, 0)),
            pl.BlockSpec((D, D), lambda i: (0, 0)),
            pl.BlockSpec((1, D), lambda i: (0, 0)),
            pl.BlockSpec((D, D), lambda i: (0, 0)),
            pl.BlockSpec((1, D), lambda i: (0, 0)),
            pl.BlockSpec((NC, nb, 16), lambda i: (0, i, 0)),
        ],
        out_specs=[
            pl.BlockSpec((nb, D), lambda i: (i, 0)),
            pl.BlockSpec((nb, D), lambda i: (i, 0)),
            pl.BlockSpec((nb, 32), lambda i: (i, 0)),
            pl.BlockSpec((nb, 32), lambda i: (i, 0)),
        ],
        out_shape=[
            jax.ShapeDtypeStruct((N_PAD, D), _f32),
            jax.ShapeDtypeStruct((N_PAD, D), _f32),
            jax.ShapeDtypeStruct((N_PAD, 32), _f32),
            jax.ShapeDtypeStruct((N_PAD, 32), _f32),
        ],
    )(features, W1, b1.reshape(1, D), W2, b2.reshape(1, D), degw)


def _deg_body(dst_ref, out_ref, deg_sp, zv, ones, dstv, sem):
    c = lax.axis_index("c")
    s = lax.axis_index("s")

    def _fz(i, _):
        zv[i, :] = jnp.zeros((16,), _f32)
        return 0
    lax.fori_loop(0, ZCH, _fz, 0)

    def _fo(i, _):
        ones[i, :] = jnp.ones((16,), _f32)
        return 0
    lax.fori_loop(0, B, _fo, 0)

    for q in range(8):
        pltpu.sync_copy(zv, deg_sp.at[pl.ds(s * RPT + q * ZCH, ZCH)])
    plsc.subcore_barrier()

    def _outer(g, _):
        b0 = c * (NBATCH // 2) + s * BPT_DEG + g * 8
        pltpu.sync_copy(dst_ref.at[pl.ds(b0, 8)], dstv)
        for j in range(8):
            pltpu.sync_copy(ones, deg_sp.at[dstv.at[j]], add=True)
        return 0
    lax.fori_loop(0, BPT_DEG // 8, _outer, 0)
    plsc.subcore_barrier()

    for q in range(8):
        r0 = s * RPT + q * ZCH
        pltpu.sync_copy(deg_sp.at[pl.ds(r0, ZCH)], zv)
        pltpu.sync_copy(zv, out_ref.at[c, pl.ds(r0, ZCH)])


def _sc_deg(dstb):
    mesh = plsc.VectorSubcoreMesh(core_axis_name="c", subcore_axis_name="s")
    return pl.kernel(
        _deg_body,
        out_type=jax.ShapeDtypeStruct((NC, N_PAD, 16), _f32),
        mesh=mesh,
        scratch_types=[
            pltpu.VMEM_SHARED((N_PAD, 16), _f32),
            pltpu.VMEM((ZCH, 16), _f32),
            pltpu.VMEM((B, 16), _f32),
            pltpu.VMEM((8, B), _i32),
            pltpu.SemaphoreType.DMA,
        ],
        compiler_params=pltpu.CompilerParams(use_tc_tiling_on_sc=False),
    )(dstb)


def _main_body(u0_ref, init0_ref, c1e_ref, c2e_ref, src2_ref, dst_ref,
               uidx_ref, h_ref, u_ref,
               agg_sp, srcv, dstv, rows, aggv, cv, iv, uidxv, sem):
    c = lax.axis_index("c")
    s = lax.axis_index("s")
    w = c * NS + s
    node0 = s * RPT

    pltpu.sync_copy(uidx_ref.at[w], uidxv)

    def _cp(k, _):
        r0 = w * RPT + k * UB
        pltpu.sync_copy(u0_ref.at[pl.ds(r0, UB)], aggv)
        pltpu.sync_copy(aggv, u_ref.at[pl.ds(r0, UB)])
        return 0
    lax.fori_loop(0, NUB, _cp, 0)

    def _initagg(b, _):
        pltpu.async_copy(init0_ref.at[uidxv.at[b]], iv, sem).wait()
        pltpu.sync_copy(iv, agg_sp.at[pl.ds(node0 + b * UB, UB)])
        return 0
    lax.fori_loop(0, NUB, _initagg, 0)
    plsc.subcore_barrier()

    def _scatter_phase():
        def _outer(g, _):
            b0 = s * BPT + g * 8
            pltpu.sync_copy(src2_ref.at[c, pl.ds(b0, 8)], srcv)
            pltpu.sync_copy(dst_ref.at[pl.ds(b0, 8)], dstv)
            for j in range(8):
                pltpu.async_copy(u_ref.at[srcv.at[j]], rows, sem).wait()
                pltpu.sync_copy(rows, agg_sp.at[dstv.at[j]], add=True)
            return 0
        lax.fori_loop(0, BPT // 8, _outer, 0)

    def _update_phase(last):
        cref = c2e_ref if last else c1e_ref
        oref = h_ref if last else u_ref

        def _ub(b, _):
            n0 = node0 + b * UB
            pltpu.sync_copy(agg_sp.at[pl.ds(n0, UB)], aggv)
            pltpu.sync_copy(cref.at[pl.ds(n0, UB)], cv)
            if not last:
                pltpu.async_copy(init0_ref.at[uidxv.at[b]], iv, sem).wait()

            def _row(r, _):
                aggv[r, pl.ds(0, 16)] = aggv[r, pl.ds(0, 16)] * cv[r, pl.ds(0, 16)]
                aggv[r, pl.ds(16, 16)] = aggv[r, pl.ds(16, 16)] * cv[r, pl.ds(16, 16)]
                return 0
            lax.fori_loop(0, UB, _row, 0)

            pltpu.sync_copy(aggv, oref.at[uidxv.at[b]])
            if not last:
                pltpu.sync_copy(iv, agg_sp.at[pl.ds(n0, UB)])
            return 0
        lax.fori_loop(0, NUB, _ub, 0)

    def _step(k, _):
        _scatter_phase()
        plsc.subcore_barrier()
        _update_phase(last=False)
        plsc.subcore_barrier()
        return 0
    lax.fori_loop(0, K - 1, _step, 0)

    _scatter_phase()
    plsc.subcore_barrier()
    _update_phase(last=True)


def _sc_main(u0v, init0v, c1e, c2e, src2, dstb, uidx):
    mesh = plsc.VectorSubcoreMesh(core_axis_name="c", subcore_axis_name="s")
    return pl.kernel(
        _main_body,
        out_type=(
            jax.ShapeDtypeStruct((2 * N_PAD, 32), _f32),
            jax.ShapeDtypeStruct((2 * N_PAD, 32), _f32),
        ),
        mesh=mesh,
        scratch_types=[
            pltpu.VMEM_SHARED((N_PAD, 32), _f32),
            pltpu.VMEM((8, B), _i32),
            pltpu.VMEM((8, B), _i32),
            pltpu.VMEM((B, 32), _f32),
            pltpu.VMEM((UB, 32), _f32),
            pltpu.VMEM((UB, 32), _f32),
            pltpu.VMEM((UB, 32), _f32),
            pltpu.VMEM((NUB, UB), _i32),
            pltpu.SemaphoreType.DMA,
        ],
        compiler_params=pltpu.CompilerParams(use_tc_tiling_on_sc=False),
    )(u0v, init0v, c1e, c2e, src2, dstb, uidx)


def kernel(features, edge_index, W1, b1, W2, b2):
    src = edge_index[0]
    dst = edge_index[1]
    pad = E_PAD - E
    pad_src = (jnp.arange(pad, dtype=_i32) * 37) % N
    pad_dst = N + (jnp.arange(pad, dtype=_i32) % (N_PAD - N))
    src_p = jnp.concatenate([src, pad_src])
    dst_p = jnp.concatenate([dst, pad_dst])
    src2 = jnp.stack([2 * src_p, 2 * src_p + 1]).reshape(NC, NBATCH, B)
    dstb = dst_p.reshape(NBATCH, B)
    nodes = jnp.arange(N_PAD, dtype=_i32).reshape(NS, NUB, UB)
    uidx = jnp.stack([2 * nodes, 2 * nodes + 1]).reshape(NW, NUB, UB)

    feats_p = jnp.pad(features, ((0, N_PAD - N), (0, 0)))
    degw = _sc_deg(dstb)
    u0, init0, c1e, c2e = _tc_prep(feats_p, W1, b1, W2, b2, degw)
    h_view, _ = _sc_main(u0.reshape(2 * N_PAD, 32), init0.reshape(2 * N_PAD, 32),
                         c1e, c2e, src2, dstb, uidx)
    return h_view.reshape(N_PAD, D)[:N]


# speedup vs baseline: 9.9987x; 1.8401x over previous
import jax
import jax.numpy as jnp
from jax import lax
from jax.experimental import pallas as pl
from jax.experimental.pallas import tpu as pltpu
from jax.experimental.pallas import tpu_sc as plsc

N = 50000
E = 800000
D = 64
K = 10
ALPHA = 0.1

NC = 2
NS = 16
NW = NC * NS
B = 128
NBATCH = 6400
E_PAD = NBATCH * B
BPT = NBATCH // NS
BPT_DEG = NBATCH // NW
N_PAD = 50176
RPT = N_PAD // NS
UB = 112
NUB = RPT // UB
ZCH = 392
GRP = 5

_f32 = jnp.float32
_i32 = jnp.int32


def _prep_body(x_ref, w1_ref, b1_ref, w2_ref, b2_ref, degw_ref,
               u0_ref, init0_ref, c1e_ref, c2e_ref):
    x = x_ref[...]
    h = jnp.dot(x, w1_ref[...].T, preferred_element_type=_f32) + b1_ref[...]
    h = jnp.maximum(h, 0.0)
    h0 = jnp.dot(h, w2_ref[...].T, preferred_element_type=_f32) + b2_ref[...]
    degw = degw_ref[...]
    deg = degw[0, :, 0] + degw[1, :, 0]
    deg = jnp.maximum(deg, 1.0)
    norm = lax.rsqrt(deg)[:, None]
    u0 = h0 * norm
    init0 = (ALPHA / (1.0 - ALPHA)) * h0 / norm
    u0_ref[0, :, :] = u0[:, :32]
    u0_ref[1, :, :] = u0[:, 32:]
    init0_ref[0, :, :] = init0[:, :32]
    init0_ref[1, :, :] = init0[:, 32:]
    nb = x.shape[0]
    c1e_ref[...] = jnp.broadcast_to((1.0 - ALPHA) * norm * norm, (nb, 32))
    c2e_ref[...] = jnp.broadcast_to((1.0 - ALPHA) * norm, (nb, 32))


def _tc_prep(features, W1, b1, W2, b2, degw):
    nb = 448
    grid = N_PAD // nb
    return pl.pallas_call(
        _prep_body,
        grid=(grid,),
        in_specs=[
            pl.BlockSpec((nb, D), lambda i: (i, 0)),
            pl.BlockSpec((D, D), lambda i: (0, 0)),
            pl.BlockSpec((1, D), lambda i: (0, 0)),
            pl.BlockSpec((D, D), lambda i: (0, 0)),
            pl.BlockSpec((1, D), lambda i: (0, 0)),
            pl.BlockSpec((NC, nb, 16), lambda i: (0, i, 0)),
        ],
        out_specs=[
            pl.BlockSpec((NC, nb, 32), lambda i: (0, i, 0)),
            pl.BlockSpec((NC, nb, 32), lambda i: (0, i, 0)),
            pl.BlockSpec((nb, 32), lambda i: (i, 0)),
            pl.BlockSpec((nb, 32), lambda i: (i, 0)),
        ],
        out_shape=[
            jax.ShapeDtypeStruct((NC, N_PAD, 32), _f32),
            jax.ShapeDtypeStruct((NC, N_PAD, 32), _f32),
            jax.ShapeDtypeStruct((N_PAD, 32), _f32),
            jax.ShapeDtypeStruct((N_PAD, 32), _f32),
        ],
    )(features, W1, b1.reshape(1, D), W2, b2.reshape(1, D), degw)


def _deg_body(dst_ref, out_ref, deg_sp, zv, ones, dstv, sem):
    c = lax.axis_index("c")
    s = lax.axis_index("s")

    def _fz(i, _):
        zv[i, :] = jnp.zeros((16,), _f32)
        return 0
    lax.fori_loop(0, ZCH, _fz, 0)

    def _fo(i, _):
        ones[i, :] = jnp.ones((16,), _f32)
        return 0
    lax.fori_loop(0, B, _fo, 0)

    for q in range(8):
        pltpu.sync_copy(zv, deg_sp.at[pl.ds(s * RPT + q * ZCH, ZCH)])
    plsc.subcore_barrier()

    def _outer(g, _):
        b0 = c * (NBATCH // 2) + s * BPT_DEG + g * 8
        pltpu.sync_copy(dst_ref.at[pl.ds(b0, 8)], dstv)
        for j in range(8):
            pltpu.sync_copy(ones, deg_sp.at[dstv.at[j]], add=True)
        return 0
    lax.fori_loop(0, BPT_DEG // 8, _outer, 0)
    plsc.subcore_barrier()

    for q in range(8):
        r0 = s * RPT + q * ZCH
        pltpu.sync_copy(deg_sp.at[pl.ds(r0, ZCH)], zv)
        pltpu.sync_copy(zv, out_ref.at[c, pl.ds(r0, ZCH)])


def _sc_deg(dstb):
    mesh = plsc.VectorSubcoreMesh(core_axis_name="c", subcore_axis_name="s")
    return pl.kernel(
        _deg_body,
        out_type=jax.ShapeDtypeStruct((NC, N_PAD, 16), _f32),
        mesh=mesh,
        scratch_types=[
            pltpu.VMEM_SHARED((N_PAD, 16), _f32),
            pltpu.VMEM((ZCH, 16), _f32),
            pltpu.VMEM((B, 16), _f32),
            pltpu.VMEM((8, B), _i32),
            pltpu.SemaphoreType.DMA,
        ],
        compiler_params=pltpu.CompilerParams(use_tc_tiling_on_sc=False),
    )(dstb)


def _main_body(u0_ref, init0_ref, c1e_ref, c2e_ref, sd_ref,
               h_ref, u_ref,
               agg_sp, sdv, rows, aggv, cv, gsem, ssem):
    c = lax.axis_index("c")
    s = lax.axis_index("s")
    w = c * NS + s
    node0 = s * RPT
    urow0 = c * N_PAD + node0

    def _cp(k, _):
        r0 = w * RPT + k * UB
        pltpu.sync_copy(u0_ref.at[pl.ds(r0, UB)], aggv)
        pltpu.sync_copy(aggv, u_ref.at[pl.ds(r0, UB)])
        return 0
    lax.fori_loop(0, NUB, _cp, 0)

    def _initagg(b, _):
        pltpu.sync_copy(init0_ref.at[pl.ds(urow0 + b * UB, UB)], cv)
        pltpu.sync_copy(cv, agg_sp.at[pl.ds(node0 + b * UB, UB)])
        return 0
    lax.fori_loop(0, NUB, _initagg, 0)
    plsc.subcore_barrier()

    def _scatter_phase():
        def _outer(g, _):
            b0 = s * BPT + g * GRP
            pltpu.sync_copy(sd_ref.at[c, pl.ds(b0, GRP)], sdv)
            gd = [pltpu.async_copy(u_ref.at[sdv.at[j, 0]], rows.at[j], gsem)
                  for j in range(GRP)]
            sds = []
            for j in range(GRP):
                gd[j].wait()
                sds.append(pltpu.async_copy(rows.at[j], agg_sp.at[sdv.at[j, 1]],
                                            ssem, add=True))
            for d in sds:
                d.wait()
            return 0
        lax.fori_loop(0, BPT // GRP, _outer, 0)

    def _update_phase(last):
        cref = c2e_ref if last else c1e_ref
        oref = h_ref if last else u_ref

        def _ub(b, _):
            n0 = node0 + b * UB
            r0 = urow0 + b * UB
            pltpu.sync_copy(agg_sp.at[pl.ds(n0, UB)], aggv)
            pltpu.sync_copy(cref.at[pl.ds(n0, UB)], cv)

            @plsc.parallel_loop(0, UB, step=1, unroll=8)
            def _row(r):
                aggv[r, pl.ds(0, 16)] = aggv[r, pl.ds(0, 16)] * cv[r, pl.ds(0, 16)]
                aggv[r, pl.ds(16, 16)] = aggv[r, pl.ds(16, 16)] * cv[r, pl.ds(16, 16)]

            pltpu.sync_copy(aggv, oref.at[pl.ds(r0, UB)])
            if not last:
                pltpu.sync_copy(init0_ref.at[pl.ds(r0, UB)], cv)
                pltpu.sync_copy(cv, agg_sp.at[pl.ds(n0, UB)])
            return 0
        lax.fori_loop(0, NUB, _ub, 0)

    def _step(k, _):
        _scatter_phase()
        plsc.subcore_barrier()
        _update_phase(last=False)
        plsc.subcore_barrier()
        return 0
    lax.fori_loop(0, K - 1, _step, 0)

    _scatter_phase()
    plsc.subcore_barrier()
    _update_phase(last=True)


def _sc_main(u0v, init0v, c1e, c2e, sdb):
    mesh = plsc.VectorSubcoreMesh(core_axis_name="c", subcore_axis_name="s")
    return pl.kernel(
        _main_body,
        out_type=(
            jax.ShapeDtypeStruct((2 * N_PAD, 32), _f32),
            jax.ShapeDtypeStruct((2 * N_PAD, 32), _f32),
        ),
        mesh=mesh,
        scratch_types=[
            pltpu.VMEM_SHARED((N_PAD, 32), _f32),
            pltpu.VMEM((GRP, 2, B), _i32),
            pltpu.VMEM((GRP, B, 32), _f32),
            pltpu.VMEM((UB, 32), _f32),
            pltpu.VMEM((UB, 32), _f32),
            pltpu.SemaphoreType.DMA,
            pltpu.SemaphoreType.DMA,
        ],
        compiler_params=pltpu.CompilerParams(use_tc_tiling_on_sc=False),
    )(u0v, init0v, c1e, c2e, sdb)


def kernel(features, edge_index, W1, b1, W2, b2):
    src = edge_index[0]
    dst = edge_index[1]
    pad = E_PAD - E
    pad_src = (jnp.arange(pad, dtype=_i32) * 37) % N
    pad_dst = N + (jnp.arange(pad, dtype=_i32) % (N_PAD - N))
    src_p = jnp.concatenate([src, pad_src])
    dst_p = jnp.concatenate([dst, pad_dst])
    src2 = jnp.stack([src_p, src_p + N_PAD]).reshape(NC, NBATCH, B)
    dstb = dst_p.reshape(NBATCH, B)
    sdb = jnp.stack([src2, jnp.broadcast_to(dstb, (NC, NBATCH, B))], axis=2)

    feats_p = jnp.pad(features, ((0, N_PAD - N), (0, 0)))
    degw = _sc_deg(dstb)
    u0, init0, c1e, c2e = _tc_prep(feats_p, W1, b1, W2, b2, degw)
    h_view, _ = _sc_main(u0.reshape(2 * N_PAD, 32), init0.reshape(2 * N_PAD, 32),
                         c1e, c2e, sdb)
    h_pair = h_view.reshape(NC, N_PAD, 32)
    return jnp.concatenate([h_pair[0, :N], h_pair[1, :N]], axis=1)


# speedup vs baseline: 10.7710x; 1.0772x over previous
import jax
import jax.numpy as jnp
from jax import lax
from jax.experimental import pallas as pl
from jax.experimental.pallas import tpu as pltpu
from jax.experimental.pallas import tpu_sc as plsc

N = 50000
E = 800000
D = 64
K = 10
ALPHA = 0.1

NC = 2
NS = 16
NW = NC * NS
B = 112
NBATCH = 7168
E_PAD = NBATCH * B
BPT = NBATCH // NS
BPT_DEG = NBATCH // NW
N_PAD = 50176
RPT = N_PAD // NS
UB = 112
NUB = RPT // UB
ZCH = 392
GRP = 8

_f32 = jnp.float32
_i32 = jnp.int32


def _prep_body(x_ref, w1_ref, b1_ref, w2_ref, b2_ref, degw_ref,
               u0_ref, init0_ref, c1e_ref, c2e_ref):
    x = x_ref[...]
    h = jnp.dot(x, w1_ref[...].T, preferred_element_type=_f32) + b1_ref[...]
    h = jnp.maximum(h, 0.0)
    h0 = jnp.dot(h, w2_ref[...].T, preferred_element_type=_f32) + b2_ref[...]
    degw = degw_ref[...]
    deg = degw[0, :, 0] + degw[1, :, 0]
    deg = jnp.maximum(deg, 1.0)
    norm = lax.rsqrt(deg)[:, None]
    u0 = h0 * norm
    init0 = (ALPHA / (1.0 - ALPHA)) * h0 / norm
    u0_ref[0, :, :] = u0[:, :32]
    u0_ref[1, :, :] = u0[:, 32:]
    init0_ref[0, :, :] = init0[:, :32]
    init0_ref[1, :, :] = init0[:, 32:]
    nb = x.shape[0]
    c1e_ref[...] = jnp.broadcast_to((1.0 - ALPHA) * norm * norm, (nb, 32))
    c2e_ref[...] = jnp.broadcast_to((1.0 - ALPHA) * norm, (nb, 32))


def _tc_prep(features, W1, b1, W2, b2, degw):
    nb = 448
    grid = N_PAD // nb
    return pl.pallas_call(
        _prep_body,
        grid=(grid,),
        in_specs=[
            pl.BlockSpec((nb, D), lambda i: (i, 0)),
            pl.BlockSpec((D, D), lambda i: (0, 0)),
            pl.BlockSpec((1, D), lambda i: (0, 0)),
            pl.BlockSpec((D, D), lambda i: (0, 0)),
            pl.BlockSpec((1, D), lambda i: (0, 0)),
            pl.BlockSpec((NC, nb, 16), lambda i: (0, i, 0)),
        ],
        out_specs=[
            pl.BlockSpec((NC, nb, 32), lambda i: (0, i, 0)),
            pl.BlockSpec((NC, nb, 32), lambda i: (0, i, 0)),
            pl.BlockSpec((nb, 32), lambda i: (i, 0)),
            pl.BlockSpec((nb, 32), lambda i: (i, 0)),
        ],
        out_shape=[
            jax.ShapeDtypeStruct((NC, N_PAD, 32), _f32),
            jax.ShapeDtypeStruct((NC, N_PAD, 32), _f32),
            jax.ShapeDtypeStruct((N_PAD, 32), _f32),
            jax.ShapeDtypeStruct((N_PAD, 32), _f32),
        ],
    )(features, W1, b1.reshape(1, D), W2, b2.reshape(1, D), degw)


def _deg_body(dst_ref, out_ref, deg_sp, zv, ones, dstv, sem):
    c = lax.axis_index("c")
    s = lax.axis_index("s")

    def _fz(i, _):
        zv[i, :] = jnp.zeros((16,), _f32)
        return 0
    lax.fori_loop(0, ZCH, _fz, 0)

    def _fo(i, _):
        ones[i, :] = jnp.ones((16,), _f32)
        return 0
    lax.fori_loop(0, B, _fo, 0)

    for q in range(8):
        pltpu.sync_copy(zv, deg_sp.at[pl.ds(s * RPT + q * ZCH, ZCH)])
    plsc.subcore_barrier()

    def _outer(g, _):
        b0 = c * (NBATCH // 2) + s * BPT_DEG + g * 8
        pltpu.sync_copy(dst_ref.at[pl.ds(b0, 8)], dstv)
        for j in range(8):
            pltpu.sync_copy(ones, deg_sp.at[dstv.at[j]], add=True)
        return 0
    lax.fori_loop(0, BPT_DEG // 8, _outer, 0)
    plsc.subcore_barrier()

    for q in range(8):
        r0 = s * RPT + q * ZCH
        pltpu.sync_copy(deg_sp.at[pl.ds(r0, ZCH)], zv)
        pltpu.sync_copy(zv, out_ref.at[c, pl.ds(r0, ZCH)])


def _sc_deg(dstb):
    mesh = plsc.VectorSubcoreMesh(core_axis_name="c", subcore_axis_name="s")
    return pl.kernel(
        _deg_body,
        out_type=jax.ShapeDtypeStruct((NC, N_PAD, 16), _f32),
        mesh=mesh,
        scratch_types=[
            pltpu.VMEM_SHARED((N_PAD, 16), _f32),
            pltpu.VMEM((ZCH, 16), _f32),
            pltpu.VMEM((B, 16), _f32),
            pltpu.VMEM((8, B), _i32),
            pltpu.SemaphoreType.DMA,
        ],
        compiler_params=pltpu.CompilerParams(use_tc_tiling_on_sc=False),
    )(dstb)


def _main_body(u0_ref, init0_ref, c1e_ref, c2e_ref, sd_ref,
               h_ref, u_ref,
               agg_sp, sdv, rows, gsem, ssem):
    c = lax.axis_index("c")
    s = lax.axis_index("s")
    w = c * NS + s
    node0 = s * RPT
    urow0 = c * N_PAD + node0
    aggv = rows.at[0]
    cv = rows.at[1]

    def _cp(k, _):
        r0 = w * RPT + k * UB
        pltpu.sync_copy(u0_ref.at[pl.ds(r0, UB)], aggv)
        pltpu.sync_copy(aggv, u_ref.at[pl.ds(r0, UB)])
        return 0
    lax.fori_loop(0, NUB, _cp, 0)

    def _initagg(b, _):
        pltpu.sync_copy(init0_ref.at[pl.ds(urow0 + b * UB, UB)], cv)
        pltpu.sync_copy(cv, agg_sp.at[pl.ds(node0 + b * UB, UB)])
        return 0
    lax.fori_loop(0, NUB, _initagg, 0)
    plsc.subcore_barrier()

    def _scatter_phase():
        def _outer(g, _):
            b0 = s * BPT + g * GRP
            pltpu.sync_copy(sd_ref.at[c, pl.ds(b0, GRP)], sdv)
            gd = [pltpu.async_copy(u_ref.at[sdv.at[j, 0]], rows.at[j], gsem)
                  for j in range(GRP)]
            sds = []
            for j in range(GRP):
                gd[j].wait()
                sds.append(pltpu.async_copy(rows.at[j], agg_sp.at[sdv.at[j, 1]],
                                            ssem, add=True))
            for d in sds:
                d.wait()
            return 0
        lax.fori_loop(0, BPT // GRP, _outer, 0)

    def _update_phase(last):
        cref = c2e_ref if last else c1e_ref
        oref = h_ref if last else u_ref

        def _ub(b, _):
            n0 = node0 + b * UB
            r0 = urow0 + b * UB
            pltpu.sync_copy(agg_sp.at[pl.ds(n0, UB)], aggv)
            pltpu.sync_copy(cref.at[pl.ds(n0, UB)], cv)

            @plsc.parallel_loop(0, UB, step=1, unroll=8)
            def _row(r):
                rows[0, r, pl.ds(0, 16)] = rows[0, r, pl.ds(0, 16)] * rows[1, r, pl.ds(0, 16)]
                rows[0, r, pl.ds(16, 16)] = rows[0, r, pl.ds(16, 16)] * rows[1, r, pl.ds(16, 16)]

            pltpu.sync_copy(aggv, oref.at[pl.ds(r0, UB)])
            if not last:
                pltpu.sync_copy(init0_ref.at[pl.ds(r0, UB)], cv)
                pltpu.sync_copy(cv, agg_sp.at[pl.ds(n0, UB)])
            return 0
        lax.fori_loop(0, NUB, _ub, 0)

    def _step(k, _):
        _scatter_phase()
        plsc.subcore_barrier()
        _update_phase(last=False)
        plsc.subcore_barrier()
        return 0
    lax.fori_loop(0, K - 1, _step, 0)

    _scatter_phase()
    plsc.subcore_barrier()
    _update_phase(last=True)


def _sc_main(u0v, init0v, c1e, c2e, sdb):
    mesh = plsc.VectorSubcoreMesh(core_axis_name="c", subcore_axis_name="s")
    return pl.kernel(
        _main_body,
        out_type=(
            jax.ShapeDtypeStruct((2 * N_PAD, 32), _f32),
            jax.ShapeDtypeStruct((2 * N_PAD, 32), _f32),
        ),
        mesh=mesh,
        scratch_types=[
            pltpu.VMEM_SHARED((N_PAD, 32), _f32),
            pltpu.VMEM((GRP, 2, B), _i32),
            pltpu.VMEM((GRP, B, 32), _f32),
            pltpu.SemaphoreType.DMA,
            pltpu.SemaphoreType.DMA,
        ],
        compiler_params=pltpu.CompilerParams(use_tc_tiling_on_sc=False),
    )(u0v, init0v, c1e, c2e, sdb)


def kernel(features, edge_index, W1, b1, W2, b2):
    src = edge_index[0]
    dst = edge_index[1]
    pad = E_PAD - E
    pad_src = (jnp.arange(pad, dtype=_i32) * 37) % N
    pad_dst = N + (jnp.arange(pad, dtype=_i32) % (N_PAD - N))
    src_p = jnp.concatenate([src, pad_src])
    dst_p = jnp.concatenate([dst, pad_dst])
    src2 = jnp.stack([src_p, src_p + N_PAD]).reshape(NC, NBATCH, B)
    dstb = dst_p.reshape(NBATCH, B)
    sdb = jnp.stack([src2, jnp.broadcast_to(dstb, (NC, NBATCH, B))], axis=2)

    feats_p = jnp.pad(features, ((0, N_PAD - N), (0, 0)))
    degw = _sc_deg(dstb)
    u0, init0, c1e, c2e = _tc_prep(feats_p, W1, b1, W2, b2, degw)
    h_view, _ = _sc_main(u0.reshape(2 * N_PAD, 32), init0.reshape(2 * N_PAD, 32),
                         c1e, c2e, sdb)
    h_pair = h_view.reshape(NC, N_PAD, 32)
    return jnp.concatenate([h_pair[0, :N], h_pair[1, :N]], axis=1)


# speedup vs baseline: 10.7771x; 1.0006x over previous
import jax
import jax.numpy as jnp
from jax import lax
from jax.experimental import pallas as pl
from jax.experimental.pallas import tpu as pltpu
from jax.experimental.pallas import tpu_sc as plsc

N = 50000
E = 800000
D = 64
K = 10
ALPHA = 0.1

NC = 2
NS = 16
NW = NC * NS
B = 112
NBATCH = 7168
E_PAD = NBATCH * B
BPT = NBATCH // NS
BPT_DEG = NBATCH // NW
N_PAD = 50176
RPT = N_PAD // NS
UB = 112
NUB = RPT // UB
ZCH = 392
GRP = 8

_f32 = jnp.float32
_i32 = jnp.int32


def _prep_body(x_ref, w1_ref, b1_ref, w2_ref, b2_ref, degw_ref,
               u0_ref, init0_ref, c1e_ref, c2e_ref):
    x = x_ref[...]
    h = jnp.dot(x, w1_ref[...].T, preferred_element_type=_f32) + b1_ref[...]
    h = jnp.maximum(h, 0.0)
    h0 = jnp.dot(h, w2_ref[...].T, preferred_element_type=_f32) + b2_ref[...]
    degw = degw_ref[...]
    deg = degw[0, :, 0] + degw[1, :, 0]
    deg = jnp.maximum(deg, 1.0)
    norm = lax.rsqrt(deg)[:, None]
    u0 = h0 * norm
    init0 = (ALPHA / (1.0 - ALPHA)) * h0 / norm
    u0_ref[0, :, :] = u0[:, :32]
    u0_ref[1, :, :] = u0[:, 32:]
    init0_ref[0, :, :] = init0[:, :32]
    init0_ref[1, :, :] = init0[:, 32:]
    nb = x.shape[0]
    c1e_ref[...] = jnp.broadcast_to((1.0 - ALPHA) * norm * norm, (nb, 32))
    c2e_ref[...] = jnp.broadcast_to((1.0 - ALPHA) * norm, (nb, 32))


def _tc_prep(features, W1, b1, W2, b2, degw):
    nb = 448
    grid = N_PAD // nb
    return pl.pallas_call(
        _prep_body,
        grid=(grid,),
        in_specs=[
            pl.BlockSpec((nb, D), lambda i: (i, 0)),
            pl.BlockSpec((D, D), lambda i: (0, 0)),
            pl.BlockSpec((1, D), lambda i: (0, 0)),
            pl.BlockSpec((D, D), lambda i: (0, 0)),
            pl.BlockSpec((1, D), lambda i: (0, 0)),
            pl.BlockSpec((NC, nb, 16), lambda i: (0, i, 0)),
        ],
        out_specs=[
            pl.BlockSpec((NC, nb, 32), lambda i: (0, i, 0)),
            pl.BlockSpec((NC, nb, 32), lambda i: (0, i, 0)),
            pl.BlockSpec((nb, 32), lambda i: (i, 0)),
            pl.BlockSpec((nb, 32), lambda i: (i, 0)),
        ],
        out_shape=[
            jax.ShapeDtypeStruct((NC, N_PAD, 32), _f32),
            jax.ShapeDtypeStruct((NC, N_PAD, 32), _f32),
            jax.ShapeDtypeStruct((N_PAD, 32), _f32),
            jax.ShapeDtypeStruct((N_PAD, 32), _f32),
        ],
    )(features, W1, b1.reshape(1, D), W2, b2.reshape(1, D), degw)


def _deg_body(dst_ref, out_ref, deg_sp, zv, ones, dstv, sem):
    c = lax.axis_index("c")
    s = lax.axis_index("s")

    def _fz(i, _):
        zv[i, :] = jnp.zeros((16,), _f32)
        return 0
    lax.fori_loop(0, ZCH, _fz, 0)

    def _fo(i, _):
        ones[i, :] = jnp.ones((16,), _f32)
        return 0
    lax.fori_loop(0, B, _fo, 0)

    for q in range(8):
        pltpu.sync_copy(zv, deg_sp.at[pl.ds(s * RPT + q * ZCH, ZCH)])
    plsc.subcore_barrier()

    def _outer(g, _):
        b0 = c * (NBATCH // 2) + s * BPT_DEG + g * 8
        pltpu.sync_copy(dst_ref.at[pl.ds(b0, 8)], dstv)
        for j in range(8):
            pltpu.sync_copy(ones, deg_sp.at[dstv.at[j]], add=True)
        return 0
    lax.fori_loop(0, BPT_DEG // 8, _outer, 0)
    plsc.subcore_barrier()

    for q in range(8):
        r0 = s * RPT + q * ZCH
        pltpu.sync_copy(deg_sp.at[pl.ds(r0, ZCH)], zv)
        pltpu.sync_copy(zv, out_ref.at[c, pl.ds(r0, ZCH)])


def _sc_deg(dstb):
    mesh = plsc.VectorSubcoreMesh(core_axis_name="c", subcore_axis_name="s")
    return pl.kernel(
        _deg_body,
        out_type=jax.ShapeDtypeStruct((NC, N_PAD, 16), _f32),
        mesh=mesh,
        scratch_types=[
            pltpu.VMEM_SHARED((N_PAD, 16), _f32),
            pltpu.VMEM((ZCH, 16), _f32),
            pltpu.VMEM((B, 16), _f32),
            pltpu.VMEM((8, B), _i32),
            pltpu.SemaphoreType.DMA,
        ],
        compiler_params=pltpu.CompilerParams(use_tc_tiling_on_sc=False),
    )(dstb)


def _main_body(u0_ref, init0_ref, c1e_ref, c2e_ref, sd_ref,
               h_ref, u_ref,
               agg_sp, sdv, rows, gsem, ssem):
    c = lax.axis_index("c")
    s = lax.axis_index("s")
    w = c * NS + s
    node0 = s * RPT
    urow0 = c * N_PAD + node0
    aggv = rows.at[0]
    cv = rows.at[1]

    def _cp(k, _):
        r0 = w * RPT + k * UB
        pltpu.sync_copy(u0_ref.at[pl.ds(r0, UB)], aggv)
        pltpu.sync_copy(aggv, u_ref.at[pl.ds(r0, UB)])
        return 0
    lax.fori_loop(0, NUB, _cp, 0)

    def _initagg(b, _):
        pltpu.sync_copy(init0_ref.at[pl.ds(urow0 + b * UB, UB)], cv)
        pltpu.sync_copy(cv, agg_sp.at[pl.ds(node0 + b * UB, UB)])
        return 0
    lax.fori_loop(0, NUB, _initagg, 0)
    plsc.subcore_barrier()

    def _scatter_phase():
        def _outer(g, _):
            b0 = s * BPT + g * GRP
            pltpu.sync_copy(sd_ref.at[c, pl.ds(b0, GRP)], sdv)
            gd = [pltpu.async_copy(u_ref.at[sdv.at[j, 0]], rows.at[j], gsem)
                  for j in range(GRP)]
            sds = []
            for j in range(GRP):
                gd[j].wait()
                sds.append(pltpu.async_copy(rows.at[j], agg_sp.at[sdv.at[j, 1]],
                                            ssem, add=True))
            for d in sds:
                d.wait()
            return 0
        lax.fori_loop(0, BPT // GRP, _outer, 0)

    def _update_phase(last):
        cref = c2e_ref if last else c1e_ref
        oref = h_ref if last else u_ref

        def _ub(b, _):
            n0 = node0 + b * UB
            r0 = urow0 + b * UB
            pltpu.sync_copy(agg_sp.at[pl.ds(n0, UB)], aggv)
            pltpu.sync_copy(cref.at[pl.ds(n0, UB)], cv)

            @plsc.parallel_loop(0, UB, step=1, unroll=8)
            def _row(r):
                rows[0, r, pl.ds(0, 16)] = rows[0, r, pl.ds(0, 16)] * rows[1, r, pl.ds(0, 16)]
                rows[0, r, pl.ds(16, 16)] = rows[0, r, pl.ds(16, 16)] * rows[1, r, pl.ds(16, 16)]

            pltpu.sync_copy(aggv, oref.at[pl.ds(r0, UB)])
            if not last:
                pltpu.sync_copy(init0_ref.at[pl.ds(r0, UB)], cv)
                pltpu.sync_copy(cv, agg_sp.at[pl.ds(n0, UB)])
            return 0
        lax.fori_loop(0, NUB, _ub, 0)

    def _step(k, _):
        with jax.named_scope("edge_scatter"):
            _scatter_phase()
        plsc.subcore_barrier()
        with jax.named_scope("node_update"):
            _update_phase(last=False)
        plsc.subcore_barrier()
        return 0
    lax.fori_loop(0, K - 1, _step, 0)

    with jax.named_scope("edge_scatter"):
        _scatter_phase()
    plsc.subcore_barrier()
    with jax.named_scope("node_update"):
        _update_phase(last=True)


def _sc_main(u0v, init0v, c1e, c2e, sdb):
    mesh = plsc.VectorSubcoreMesh(core_axis_name="c", subcore_axis_name="s")
    return pl.kernel(
        _main_body,
        out_type=(
            jax.ShapeDtypeStruct((2 * N_PAD, 32), _f32),
            jax.ShapeDtypeStruct((2 * N_PAD, 32), _f32),
        ),
        mesh=mesh,
        scratch_types=[
            pltpu.VMEM_SHARED((N_PAD, 32), _f32),
            pltpu.VMEM((GRP, 2, B), _i32),
            pltpu.VMEM((GRP, B, 32), _f32),
            pltpu.SemaphoreType.DMA,
            pltpu.SemaphoreType.DMA,
        ],
        compiler_params=pltpu.CompilerParams(use_tc_tiling_on_sc=False),
    )(u0v, init0v, c1e, c2e, sdb)


def kernel(features, edge_index, W1, b1, W2, b2):
    src = edge_index[0]
    dst = edge_index[1]
    pad = E_PAD - E
    pad_src = (jnp.arange(pad, dtype=_i32) * 37) % N
    pad_dst = N + (jnp.arange(pad, dtype=_i32) % (N_PAD - N))
    src_p = jnp.concatenate([src, pad_src])
    dst_p = jnp.concatenate([dst, pad_dst])
    src2 = jnp.stack([src_p, src_p + N_PAD]).reshape(NC, NBATCH, B)
    dstb = dst_p.reshape(NBATCH, B)
    sdb = jnp.stack([src2, jnp.broadcast_to(dstb, (NC, NBATCH, B))], axis=2)

    feats_p = jnp.pad(features, ((0, N_PAD - N), (0, 0)))
    degw = _sc_deg(dstb)
    u0, init0, c1e, c2e = _tc_prep(feats_p, W1, b1, W2, b2, degw)
    h_view, _ = _sc_main(u0.reshape(2 * N_PAD, 32), init0.reshape(2 * N_PAD, 32),
                         c1e, c2e, sdb)
    h_pair = h_view.reshape(NC, N_PAD, 32)
    return jnp.concatenate([h_pair[0, :N], h_pair[1, :N]], axis=1)


# speedup vs baseline: 12.4732x; 1.1574x over previous
import jax
import jax.numpy as jnp
from jax import lax
from jax.experimental import pallas as pl
from jax.experimental.pallas import tpu as pltpu
from jax.experimental.pallas import tpu_sc as plsc

N = 50000
E = 800000
D = 64
K = 10
ALPHA = 0.1

NC = 2
NS = 16
NW = NC * NS
B = 112
NBATCH = 7168
E_PAD = NBATCH * B
BPT = NBATCH // NS
BPT_DEG = NBATCH // NW
N_PAD = 50176
RPT = N_PAD // NS
UB = 448
NUB = RPT // UB
ZCH = 392
GRP = 8

_f32 = jnp.float32
_i32 = jnp.int32


def _prep_body(x_ref, w1_ref, b1_ref, w2_ref, b2_ref, degw_ref,
               u0_ref, init0_ref, c1e_ref, c2e_ref):
    x = x_ref[...]
    h = jnp.dot(x, w1_ref[...].T, preferred_element_type=_f32) + b1_ref[...]
    h = jnp.maximum(h, 0.0)
    h0 = jnp.dot(h, w2_ref[...].T, preferred_element_type=_f32) + b2_ref[...]
    degw = degw_ref[...]
    deg = degw[0, :, 0] + degw[1, :, 0]
    deg = jnp.maximum(deg, 1.0)
    norm = lax.rsqrt(deg)[:, None]
    u0 = h0 * norm
    init0 = (ALPHA / (1.0 - ALPHA)) * h0 / norm
    u0_ref[0, :, :] = u0[:, :32]
    u0_ref[1, :, :] = u0[:, 32:]
    init0_ref[0, :, :] = init0[:, :32]
    init0_ref[1, :, :] = init0[:, 32:]
    nb = x.shape[0]
    c1e_ref[...] = jnp.broadcast_to((1.0 - ALPHA) * norm * norm, (nb, 32))
    c2e_ref[...] = jnp.broadcast_to((1.0 - ALPHA) * norm, (nb, 32))


def _tc_prep(features, W1, b1, W2, b2, degw):
    nb = 448
    grid = N_PAD // nb
    return pl.pallas_call(
        _prep_body,
        grid=(grid,),
        in_specs=[
            pl.BlockSpec((nb, D), lambda i: (i, 0)),
            pl.BlockSpec((D, D), lambda i: (0, 0)),
            pl.BlockSpec((1, D), lambda i: (0, 0)),
            pl.BlockSpec((D, D), lambda i: (0, 0)),
            pl.BlockSpec((1, D), lambda i: (0, 0)),
            pl.BlockSpec((NC, nb, 16), lambda i: (0, i, 0)),
        ],
        out_specs=[
            pl.BlockSpec((NC, nb, 32), lambda i: (0, i, 0)),
            pl.BlockSpec((NC, nb, 32), lambda i: (0, i, 0)),
            pl.BlockSpec((nb, 32), lambda i: (i, 0)),
            pl.BlockSpec((nb, 32), lambda i: (i, 0)),
        ],
        out_shape=[
            jax.ShapeDtypeStruct((NC, N_PAD, 32), _f32),
            jax.ShapeDtypeStruct((NC, N_PAD, 32), _f32),
            jax.ShapeDtypeStruct((N_PAD, 32), _f32),
            jax.ShapeDtypeStruct((N_PAD, 32), _f32),
        ],
    )(features, W1, b1.reshape(1, D), W2, b2.reshape(1, D), degw)


def _deg_body(dst_ref, out_ref, deg_sp, zv, ones, dstv, sem):
    c = lax.axis_index("c")
    s = lax.axis_index("s")

    def _fz(i, _):
        zv[i, :] = jnp.zeros((16,), _f32)
        return 0
    lax.fori_loop(0, ZCH, _fz, 0)

    def _fo(i, _):
        ones[i, :] = jnp.ones((16,), _f32)
        return 0
    lax.fori_loop(0, B, _fo, 0)

    for q in range(8):
        pltpu.sync_copy(zv, deg_sp.at[pl.ds(s * RPT + q * ZCH, ZCH)])
    plsc.subcore_barrier()

    def _outer(g, _):
        b0 = c * (NBATCH // 2) + s * BPT_DEG + g * 8
        pltpu.sync_copy(dst_ref.at[pl.ds(b0, 8)], dstv)
        for j in range(8):
            pltpu.sync_copy(ones, deg_sp.at[dstv.at[j]], add=True)
        return 0
    lax.fori_loop(0, BPT_DEG // 8, _outer, 0)
    plsc.subcore_barrier()

    for q in range(8):
        r0 = s * RPT + q * ZCH
        pltpu.sync_copy(deg_sp.at[pl.ds(r0, ZCH)], zv)
        pltpu.sync_copy(zv, out_ref.at[c, pl.ds(r0, ZCH)])


def _sc_deg(dstb):
    mesh = plsc.VectorSubcoreMesh(core_axis_name="c", subcore_axis_name="s")
    return pl.kernel(
        _deg_body,
        out_type=jax.ShapeDtypeStruct((NC, N_PAD, 16), _f32),
        mesh=mesh,
        scratch_types=[
            pltpu.VMEM_SHARED((N_PAD, 16), _f32),
            pltpu.VMEM((ZCH, 16), _f32),
            pltpu.VMEM((B, 16), _f32),
            pltpu.VMEM((8, B), _i32),
            pltpu.SemaphoreType.DMA,
        ],
        compiler_params=pltpu.CompilerParams(use_tc_tiling_on_sc=False),
    )(dstb)


def _main_body(u0_ref, init0_ref, c1e_ref, c2e_ref, sd_ref,
               h_ref, u_ref,
               agg_sp, sdv, rows, gsem, ssem):
    c = lax.axis_index("c")
    s = lax.axis_index("s")
    w = c * NS + s
    node0 = s * RPT
    urow0 = c * N_PAD + node0
    u_view = u_ref.at[pl.ds(c * N_PAD, N_PAD)]
    aggv = rows.at[pl.ds(0, UB)]
    cv = rows.at[pl.ds(UB, UB)]

    def _cp(k, _):
        r0 = w * RPT + k * UB
        pltpu.sync_copy(u0_ref.at[pl.ds(r0, UB)], aggv)
        pltpu.sync_copy(aggv, u_ref.at[pl.ds(r0, UB)])
        return 0
    lax.fori_loop(0, NUB, _cp, 0)

    def _initagg(b, _):
        pltpu.sync_copy(init0_ref.at[pl.ds(urow0 + b * UB, UB)], cv)
        pltpu.sync_copy(cv, agg_sp.at[pl.ds(node0 + b * UB, UB)])
        return 0
    lax.fori_loop(0, NUB, _initagg, 0)
    plsc.subcore_barrier()

    def _scatter_phase():
        def _outer(g, _):
            b0 = s * BPT + g * GRP
            pltpu.sync_copy(sd_ref.at[pl.ds(b0, GRP)], sdv)
            gd = [pltpu.async_copy(u_view.at[sdv.at[j, 0]],
                                   rows.at[pl.ds(j * B, B)], gsem)
                  for j in range(GRP)]
            sds = []
            for j in range(GRP):
                gd[j].wait()
                sds.append(pltpu.async_copy(rows.at[pl.ds(j * B, B)],
                                            agg_sp.at[sdv.at[j, 1]],
                                            ssem, add=True))
            for d in sds:
                d.wait()
            return 0
        lax.fori_loop(0, BPT // GRP, _outer, 0)

    def _update_phase(last):
        cref = c2e_ref if last else c1e_ref
        oref = h_ref if last else u_ref

        def _ub(b, _):
            n0 = node0 + b * UB
            r0 = urow0 + b * UB
            pltpu.sync_copy(agg_sp.at[pl.ds(n0, UB)], aggv)
            pltpu.sync_copy(cref.at[pl.ds(n0, UB)], cv)

            @plsc.parallel_loop(0, UB, step=1, unroll=8)
            def _row(r):
                rows[r, pl.ds(0, 16)] = rows[r, pl.ds(0, 16)] * rows[UB + r, pl.ds(0, 16)]
                rows[r, pl.ds(16, 16)] = rows[r, pl.ds(16, 16)] * rows[UB + r, pl.ds(16, 16)]

            pltpu.sync_copy(aggv, oref.at[pl.ds(r0, UB)])
            if not last:
                pltpu.sync_copy(init0_ref.at[pl.ds(r0, UB)], cv)
                pltpu.sync_copy(cv, agg_sp.at[pl.ds(n0, UB)])
            return 0
        lax.fori_loop(0, NUB, _ub, 0)

    def _step(k, _):
        with jax.named_scope("edge_scatter"):
            _scatter_phase()
        plsc.subcore_barrier()
        with jax.named_scope("node_update"):
            _update_phase(last=False)
        plsc.subcore_barrier()
        return 0
    lax.fori_loop(0, K - 1, _step, 0)

    with jax.named_scope("edge_scatter"):
        _scatter_phase()
    plsc.subcore_barrier()
    with jax.named_scope("node_update"):
        _update_phase(last=True)


def _sc_main(u0v, init0v, c1e, c2e, sdb):
    mesh = plsc.VectorSubcoreMesh(core_axis_name="c", subcore_axis_name="s")
    return pl.kernel(
        _main_body,
        out_type=(
            jax.ShapeDtypeStruct((2 * N_PAD, 32), _f32),
            jax.ShapeDtypeStruct((2 * N_PAD, 32), _f32),
        ),
        mesh=mesh,
        scratch_types=[
            pltpu.VMEM_SHARED((N_PAD, 32), _f32),
            pltpu.VMEM((GRP, 2, B), _i32),
            pltpu.VMEM((GRP * B, 32), _f32),
            pltpu.SemaphoreType.DMA,
            pltpu.SemaphoreType.DMA,
        ],
        compiler_params=pltpu.CompilerParams(use_tc_tiling_on_sc=False),
    )(u0v, init0v, c1e, c2e, sdb)


def kernel(features, edge_index, W1, b1, W2, b2):
    src = edge_index[0]
    dst = edge_index[1]
    pad = E_PAD - E
    pad_src = (jnp.arange(pad, dtype=_i32) * 37) % N
    pad_dst = N + (jnp.arange(pad, dtype=_i32) % (N_PAD - N))
    src_p = jnp.concatenate([src, pad_src])
    dst_p = jnp.concatenate([dst, pad_dst])
    dstb = dst_p.reshape(NBATCH, B)
    sdb = jnp.stack([src_p.reshape(NBATCH, B), dstb], axis=1)

    feats_p = jnp.pad(features, ((0, N_PAD - N), (0, 0)))
    degw = _sc_deg(dstb)
    u0, init0, c1e, c2e = _tc_prep(feats_p, W1, b1, W2, b2, degw)
    h_view, _ = _sc_main(u0.reshape(2 * N_PAD, 32), init0.reshape(2 * N_PAD, 32),
                         c1e, c2e, sdb)
    h_pair = h_view.reshape(NC, N_PAD, 32)
    return jnp.concatenate([h_pair[0, :N], h_pair[1, :N]], axis=1)


# speedup vs baseline: 12.7358x; 1.0211x over previous
import jax
import jax.numpy as jnp
from jax import lax
from jax.experimental import pallas as pl
from jax.experimental.pallas import tpu as pltpu
from jax.experimental.pallas import tpu_sc as plsc

N = 50000
E = 800000
D = 64
K = 10
ALPHA = 0.1

NC = 2
NS = 16
NW = NC * NS
B = 112
NBATCH = 7168
E_PAD = NBATCH * B
BPT = NBATCH // NS
BPT_DEG = NBATCH // NW
N_PAD = 50176
RPT = N_PAD // NS
UB = 448
NUB = RPT // UB
ZCH = 392
GRP = 8

_f32 = jnp.float32
_i32 = jnp.int32


def _prep_body(x_ref, w1_ref, b1_ref, w2_ref, b2_ref, degw_ref,
               u0_ref, init0_ref, c1e_ref, c2e_ref):
    x = x_ref[...]
    h = jnp.dot(x, w1_ref[...].T, preferred_element_type=_f32) + b1_ref[...]
    h = jnp.maximum(h, 0.0)
    h0 = jnp.dot(h, w2_ref[...].T, preferred_element_type=_f32) + b2_ref[...]
    degw = degw_ref[...]
    deg = degw[0, :, 0] + degw[1, :, 0]
    deg = jnp.maximum(deg, 1.0)
    norm = lax.rsqrt(deg)[:, None]
    u0 = h0 * norm
    init0 = (ALPHA / (1.0 - ALPHA)) * h0 / norm
    u0_ref[0, :, :] = u0[:, :32]
    u0_ref[1, :, :] = u0[:, 32:]
    init0_ref[0, :, :] = init0[:, :32]
    init0_ref[1, :, :] = init0[:, 32:]
    nb = x.shape[0]
    c1e_ref[...] = jnp.broadcast_to((1.0 - ALPHA) * norm * norm, (nb, 32))
    c2e_ref[...] = jnp.broadcast_to((1.0 - ALPHA) * norm, (nb, 32))


def _tc_prep(features, W1, b1, W2, b2, degw):
    nb = 3136
    grid = N_PAD // nb
    return pl.pallas_call(
        _prep_body,
        grid=(grid,),
        in_specs=[
            pl.BlockSpec((nb, D), lambda i: (i, 0)),
            pl.BlockSpec((D, D), lambda i: (0, 0)),
            pl.BlockSpec((1, D), lambda i: (0, 0)),
            pl.BlockSpec((D, D), lambda i: (0, 0)),
            pl.BlockSpec((1, D), lambda i: (0, 0)),
            pl.BlockSpec((NC, nb, 16), lambda i: (0, i, 0)),
        ],
        out_specs=[
            pl.BlockSpec((NC, nb, 32), lambda i: (0, i, 0)),
            pl.BlockSpec((NC, nb, 32), lambda i: (0, i, 0)),
            pl.BlockSpec((nb, 32), lambda i: (i, 0)),
            pl.BlockSpec((nb, 32), lambda i: (i, 0)),
        ],
        out_shape=[
            jax.ShapeDtypeStruct((NC, N_PAD, 32), _f32),
            jax.ShapeDtypeStruct((NC, N_PAD, 32), _f32),
            jax.ShapeDtypeStruct((N_PAD, 32), _f32),
            jax.ShapeDtypeStruct((N_PAD, 32), _f32),
        ],
    )(features, W1, b1.reshape(1, D), W2, b2.reshape(1, D), degw)


def _deg_body(dst_ref, out_ref, deg_sp, zv, ones, dstv, sem):
    c = lax.axis_index("c")
    s = lax.axis_index("s")

    def _fz(i, _):
        zv[i, :] = jnp.zeros((16,), _f32)
        return 0
    lax.fori_loop(0, ZCH, _fz, 0)

    def _fo(i, _):
        ones[i, :] = jnp.ones((16,), _f32)
        return 0
    lax.fori_loop(0, B, _fo, 0)

    for q in range(8):
        pltpu.sync_copy(zv, deg_sp.at[pl.ds(s * RPT + q * ZCH, ZCH)])
    plsc.subcore_barrier()

    def _outer(g, _):
        b0 = c * (NBATCH // 2) + s * BPT_DEG + g * 8
        pltpu.sync_copy(dst_ref.at[pl.ds(b0, 8)], dstv)
        ds_l = [pltpu.async_copy(ones, deg_sp.at[dstv.at[j]], sem, add=True)
                for j in range(8)]
        for d in ds_l:
            d.wait()
        return 0
    lax.fori_loop(0, BPT_DEG // 8, _outer, 0)
    plsc.subcore_barrier()

    for q in range(8):
        r0 = s * RPT + q * ZCH
        pltpu.sync_copy(deg_sp.at[pl.ds(r0, ZCH)], zv)
        pltpu.sync_copy(zv, out_ref.at[c, pl.ds(r0, ZCH)])


def _sc_deg(dstb):
    mesh = plsc.VectorSubcoreMesh(core_axis_name="c", subcore_axis_name="s")
    return pl.kernel(
        _deg_body,
        out_type=jax.ShapeDtypeStruct((NC, N_PAD, 16), _f32),
        mesh=mesh,
        scratch_types=[
            pltpu.VMEM_SHARED((N_PAD, 16), _f32),
            pltpu.VMEM((ZCH, 16), _f32),
            pltpu.VMEM((B, 16), _f32),
            pltpu.VMEM((8, B), _i32),
            pltpu.SemaphoreType.DMA,
        ],
        compiler_params=pltpu.CompilerParams(use_tc_tiling_on_sc=False),
    )(dstb)


def _main_body(u0_ref, init0_ref, c1e_ref, c2e_ref, sd_ref,
               h_ref, u_ref,
               agg_sp, sdv, rows, gsem, ssem):
    c = lax.axis_index("c")
    s = lax.axis_index("s")
    w = c * NS + s
    node0 = s * RPT
    u_view = u_ref.at[c]
    aggv = rows.at[pl.ds(0, UB)]
    cv = rows.at[pl.ds(UB, UB)]

    def _cp(k, _):
        r0 = node0 + k * UB
        pltpu.sync_copy(u0_ref.at[c, pl.ds(r0, UB)], aggv)
        pltpu.sync_copy(aggv, u_ref.at[c, pl.ds(r0, UB)])
        return 0
    lax.fori_loop(0, NUB, _cp, 0)

    def _initagg(b, _):
        pltpu.sync_copy(init0_ref.at[c, pl.ds(node0 + b * UB, UB)], cv)
        pltpu.sync_copy(cv, agg_sp.at[pl.ds(node0 + b * UB, UB)])
        return 0
    lax.fori_loop(0, NUB, _initagg, 0)
    plsc.subcore_barrier()

    def _scatter_phase():
        def _outer(g, _):
            b0 = s * BPT + g * GRP
            pltpu.sync_copy(sd_ref.at[pl.ds(b0, GRP)], sdv)
            gd = [pltpu.async_copy(u_view.at[sdv.at[j, 0]],
                                   rows.at[pl.ds(j * B, B)], gsem)
                  for j in range(GRP)]
            sds = []
            for j in range(GRP):
                gd[j].wait()
                sds.append(pltpu.async_copy(rows.at[pl.ds(j * B, B)],
                                            agg_sp.at[sdv.at[j, 1]],
                                            ssem, add=True))
            for d in sds:
                d.wait()
            return 0
        lax.fori_loop(0, BPT // GRP, _outer, 0)

    def _update_phase(last):
        cref = c2e_ref if last else c1e_ref
        oref = h_ref if last else u_ref

        def _ub(b, _):
            n0 = node0 + b * UB
            pltpu.sync_copy(agg_sp.at[pl.ds(n0, UB)], aggv)
            pltpu.sync_copy(cref.at[pl.ds(n0, UB)], cv)

            @plsc.parallel_loop(0, UB, step=1, unroll=8)
            def _row(r):
                rows[r, pl.ds(0, 16)] = rows[r, pl.ds(0, 16)] * rows[UB + r, pl.ds(0, 16)]
                rows[r, pl.ds(16, 16)] = rows[r, pl.ds(16, 16)] * rows[UB + r, pl.ds(16, 16)]

            pltpu.sync_copy(aggv, oref.at[c, pl.ds(n0, UB)])
            if not last:
                pltpu.sync_copy(init0_ref.at[c, pl.ds(n0, UB)], cv)
                pltpu.sync_copy(cv, agg_sp.at[pl.ds(n0, UB)])
            return 0
        lax.fori_loop(0, NUB, _ub, 0)

    def _step(k, _):
        with jax.named_scope("edge_scatter"):
            _scatter_phase()
        plsc.subcore_barrier()
        with jax.named_scope("node_update"):
            _update_phase(last=False)
        plsc.subcore_barrier()
        return 0
    lax.fori_loop(0, K - 1, _step, 0)

    with jax.named_scope("edge_scatter"):
        _scatter_phase()
    plsc.subcore_barrier()
    with jax.named_scope("node_update"):
        _update_phase(last=True)


def _sc_main(u0v, init0v, c1e, c2e, sdb):
    mesh = plsc.VectorSubcoreMesh(core_axis_name="c", subcore_axis_name="s")
    return pl.kernel(
        _main_body,
        out_type=(
            jax.ShapeDtypeStruct((NC, N_PAD, 32), _f32),
            jax.ShapeDtypeStruct((NC, N_PAD, 32), _f32),
        ),
        mesh=mesh,
        scratch_types=[
            pltpu.VMEM_SHARED((N_PAD, 32), _f32),
            pltpu.VMEM((GRP, 2, B), _i32),
            pltpu.VMEM((GRP * B, 32), _f32),
            pltpu.SemaphoreType.DMA,
            pltpu.SemaphoreType.DMA,
        ],
        compiler_params=pltpu.CompilerParams(use_tc_tiling_on_sc=False),
    )(u0v, init0v, c1e, c2e, sdb)


def kernel(features, edge_index, W1, b1, W2, b2):
    src = edge_index[0]
    dst = edge_index[1]
    pad = E_PAD - E
    pad_src = (jnp.arange(pad, dtype=_i32) * 37) % N
    pad_dst = N + (jnp.arange(pad, dtype=_i32) % (N_PAD - N))
    src_p = jnp.concatenate([src, pad_src])
    dst_p = jnp.concatenate([dst, pad_dst])
    dstb = dst_p.reshape(NBATCH, B)
    sdb = jnp.stack([src_p.reshape(NBATCH, B), dstb], axis=1)

    feats_p = jnp.pad(features, ((0, N_PAD - N), (0, 0)))
    degw = _sc_deg(dstb)
    u0, init0, c1e, c2e = _tc_prep(feats_p, W1, b1, W2, b2, degw)
    h_pair, _ = _sc_main(u0, init0, c1e, c2e, sdb)
    return jnp.concatenate([h_pair[0, :N], h_pair[1, :N]], axis=1)
